# baseline (speedup 1.0000x reference)
import functools

import jax
import jax.numpy as jnp
from jax import lax
from jax.experimental import pallas as pl
from jax.experimental.pallas import tpu as pltpu
from jax.experimental.pallas import tpu_sc as plsc

_CH = 128
_SCK = 256


def _make_kernel(B, S, H, V):
    info = plsc.get_sparse_core_info()
    NC, NS = info.num_cores, info.num_subcores
    NW = NC * NS
    TOK = B * S
    TPW = TOK // NW
    CH = _CH
    SCK = _SCK
    NSC = TPW // SCK
    NCH = TPW // CH
    G = H // 16
    NB = 3

    mesh = plsc.VectorSubcoreMesh(core_axis_name="c", subcore_axis_name="s")

    @functools.partial(
        pl.kernel,
        out_type=jax.ShapeDtypeStruct((TOK // CH, CH, H), jnp.float32),
        mesh=mesh,
        compiler_params=pltpu.CompilerParams(use_tc_tiling_on_sc=False),
        scratch_types=[
            pltpu.VMEM((NCH, CH), jnp.int32),
            pltpu.VMEM((TPW,), jnp.int32),
            pltpu.VMEM((2, CH, H), jnp.float32),
            pltpu.VMEM((2, CH, H), jnp.float32),
            pltpu.VMEM((2, CH, H), jnp.float32),
            pltpu.VMEM((S, H), jnp.float32),
            pltpu.VMEM((2 * S, H), jnp.float32),
            pltpu.VMEM((TPW,), jnp.int32),
            pltpu.VMEM((2, H), jnp.float32),
            pltpu.SemaphoreType.DMA,
            pltpu.SemaphoreType.DMA,
            pltpu.SemaphoreType.DMA,
            pltpu.SemaphoreType.DMA,
            pltpu.SemaphoreType.DMA,
            pltpu.SemaphoreType.DMA,
        ],
    )
    def k(in_hbm, maskf_hbm, emb_hbm, pos_hbm, mt_hbm, out_hbm,
          tall, mall, erow0, erow1, erow2, posv, posm, cidx, mtv,
          semg0, semg1, semg2, semo0, semo1, semo2):
        wid = lax.axis_index("s") * NC + lax.axis_index("c")
        pltpu.sync_copy(pos_hbm, posv)
        pltpu.sync_copy(mt_hbm, mtv)
        pltpu.sync_copy(in_hbm.at[pl.ds(wid * NCH, NCH), :], tall)
        pltpu.sync_copy(maskf_hbm.at[pl.ds(wid * TPW, TPW)], mall)

        mt0 = [mtv[0, pl.ds(j * 16, 16)] for j in range(G)]
        mt1 = [mtv[1, pl.ds(j * 16, 16)] for j in range(G)]

        def pos_prep(s, carry):
            for j in range(G):
                sl = pl.ds(j * 16, 16)
                v = posv[s, sl]
                posm[2 * s, sl] = v + mt0[j]
                posm[2 * s + 1, sl] = v + mt1[j]
            return carry

        lax.fori_loop(0, S, pos_prep, 0)

        lane = jnp.arange(16, dtype=jnp.int32)

        def cidx_prep(ii, carry):
            sl = pl.ds(ii * 16, 16)
            cidx[sl] = 2 * lax.rem(lane + ii * 16, S) + mall[sl]
            return carry

        lax.fori_loop(0, TPW // 16, cidx_prep, 0)

        erow = (erow0, erow1, erow2)
        semg = (semg0, semg1, semg2)
        semo = (semo0, semo1, semo2)

        def fire_gather(u, p):
            pltpu.async_copy(emb_hbm.at[tall.at[2 * u]],
                             erow[p].at[0], semg[p])
            pltpu.async_copy(emb_hbm.at[tall.at[2 * u + 1]],
                             erow[p].at[1], semg[p])

        def out_slice(u):
            return out_hbm.at[pl.ds(wid * NCH + 2 * u, 2), :, :]

        def drain_gather(p):
            pltpu.make_async_copy(out_slice(0), erow[p], semg[p]).wait()

        def drain_out(u, p):
            pltpu.make_async_copy(erow[p], out_slice(u), semo[p]).wait()

        def compute(u, p):
            def g_body(g, carry):
                toff = u * SCK + g * 16
                cvec = cidx[pl.ds(toff, 16)]
                for q in range(16):
                    t = g * 16 + q
                    h = t // CH
                    r = t % CH
                    ci = cvec[q]
                    for j in range(G):
                        sl = pl.ds(j * 16, 16)
                        erow[p][h, r, sl] = erow[p][h, r, sl] + posm[ci, sl]
                return carry

            lax.fori_loop(0, SCK // 16, g_body, 0)

        def stage(u, p):
            q = (p + NB - 1) % NB

            @pl.when(u < NSC)
            def _():
                @pl.when(u + NB - 1 < NSC)
                def _():
                    @pl.when(u >= 1)
                    def _():
                        drain_out(u - 1, q)
                    fire_gather(u + NB - 1, q)

                drain_gather(p)
                compute(u, p)
                pltpu.async_copy(erow[p], out_slice(u), semo[p])

        def ring_body(ii, carry):
            for r in range(NB):
                stage(NB * ii + r, r)
            return carry

        for r in range(NB - 1):
            fire_gather(r, r)
        lax.fori_loop(0, (NSC + NB - 1) // NB, ring_body, 0)
        for u in range(NSC - NB, NSC):
            drain_out(u, u % NB)

    return k


def kernel(Input, mask, emb_table, pos_table, mask_table):
    B, S = Input.shape
    V, H = emb_table.shape
    k = _make_kernel(B, S, H, V)
    out = k(Input.reshape(-1, _CH), mask.reshape(-1), emb_table,
            pos_table[:S], mask_table)
    return out.reshape(B, S, H)


# speedup vs baseline: 1.3034x; 1.3034x over previous
import functools

import jax
import jax.numpy as jnp
from jax import lax
from jax.experimental import pallas as pl
from jax.experimental.pallas import tpu as pltpu
from jax.experimental.pallas import tpu_sc as plsc

_CH = 128
_SCK = 256


def _make_kernel(B, S, H, V):
    info = plsc.get_sparse_core_info()
    NC, NS = info.num_cores, info.num_subcores
    NW = NC * NS
    TOK = B * S
    TPW = TOK // NW
    CH = _CH
    SCK = _SCK
    NSC = TPW // SCK
    NCH = TPW // CH
    G = H // 16
    NB = 3

    mesh = plsc.VectorSubcoreMesh(core_axis_name="c", subcore_axis_name="s")

    @functools.partial(
        pl.kernel,
        out_type=jax.ShapeDtypeStruct((TOK // CH, CH, H), jnp.float32),
        mesh=mesh,
        compiler_params=pltpu.CompilerParams(use_tc_tiling_on_sc=False),
        scratch_types=[
            pltpu.VMEM((NCH, CH), jnp.int32),
            pltpu.VMEM((TPW,), jnp.int32),
            pltpu.VMEM((2, CH, H), jnp.float32),
            pltpu.VMEM((2, CH, H), jnp.float32),
            pltpu.VMEM((2, CH, H), jnp.float32),
            pltpu.VMEM((S, H), jnp.float32),
            pltpu.VMEM((2 * S, H), jnp.float32),
            pltpu.VMEM((TPW + 16,), jnp.int32),
            pltpu.VMEM((2, H), jnp.float32),
            pltpu.SemaphoreType.DMA,
            pltpu.SemaphoreType.DMA,
            pltpu.SemaphoreType.DMA,
            pltpu.SemaphoreType.DMA,
            pltpu.SemaphoreType.DMA,
            pltpu.SemaphoreType.DMA,
        ],
    )
    def k(in_hbm, maskf_hbm, emb_hbm, pos_hbm, mt_hbm, out_hbm,
          tall, mall, erow0, erow1, erow2, posv, posm, cidx, mtv,
          semg0, semg1, semg2, semo0, semo1, semo2):
        wid = lax.axis_index("s") * NC + lax.axis_index("c")
        pltpu.sync_copy(pos_hbm, posv)
        pltpu.sync_copy(mt_hbm, mtv)
        pltpu.sync_copy(in_hbm.at[pl.ds(wid * NCH, NCH), :], tall)
        pltpu.sync_copy(maskf_hbm.at[pl.ds(wid * TPW, TPW)], mall)

        mt0 = [mtv[0, pl.ds(j * 16, 16)] for j in range(G)]
        mt1 = [mtv[1, pl.ds(j * 16, 16)] for j in range(G)]

        def pos_prep(s, carry):
            for j in range(G):
                sl = pl.ds(j * 16, 16)
                v = posv[s, sl]
                posm[2 * s, sl] = v + mt0[j]
                posm[2 * s + 1, sl] = v + mt1[j]
            return carry

        lax.fori_loop(0, S, pos_prep, 0)

        lane = jnp.arange(16, dtype=jnp.int32)

        def cidx_prep(ii, carry):
            sl = pl.ds(ii * 16, 16)
            cidx[sl] = 2 * lax.rem(lane + ii * 16, S) + mall[sl]
            return carry

        lax.fori_loop(0, TPW // 16, cidx_prep, 0)

        erow = (erow0, erow1, erow2)
        semg = (semg0, semg1, semg2)
        semo = (semo0, semo1, semo2)

        def fire_gather(u, p):
            pltpu.async_copy(emb_hbm.at[tall.at[2 * u]],
                             erow[p].at[0], semg[p])
            pltpu.async_copy(emb_hbm.at[tall.at[2 * u + 1]],
                             erow[p].at[1], semg[p])

        def out_slice(u):
            return out_hbm.at[pl.ds(wid * NCH + 2 * u, 2), :, :]

        def drain_gather(p):
            pltpu.make_async_copy(out_slice(0), erow[p], semg[p]).wait()

        def drain_out(u, p):
            pltpu.make_async_copy(erow[p], out_slice(u), semo[p]).wait()

        def compute(u, p):
            @plsc.parallel_loop(0, SCK, 1, unroll=4)
            def body(t):
                ci = cidx[pl.ds(u * SCK + t, 16)][0]
                h = t // CH
                r = t % CH
                for j in range(G):
                    sl = pl.ds(j * 16, 16)
                    erow[p][h, r, sl] = erow[p][h, r, sl] + posm[ci, sl]

        def stage(u, p):
            q = (p + NB - 1) % NB

            @pl.when(u < NSC)
            def _():
                @pl.when(u + NB - 1 < NSC)
                def _():
                    @pl.when(u >= 1)
                    def _():
                        drain_out(u - 1, q)
                    fire_gather(u + NB - 1, q)

                drain_gather(p)
                compute(u, p)
                pltpu.async_copy(erow[p], out_slice(u), semo[p])

        def ring_body(ii, carry):
            for r in range(NB):
                stage(NB * ii + r, r)
            return carry

        for r in range(NB - 1):
            fire_gather(r, r)
        lax.fori_loop(0, (NSC + NB - 1) // NB, ring_body, 0)
        for u in range(NSC - NB, NSC):
            drain_out(u, u % NB)

    return k


def kernel(Input, mask, emb_table, pos_table, mask_table):
    B, S = Input.shape
    V, H = emb_table.shape
    k = _make_kernel(B, S, H, V)
    out = k(Input.reshape(-1, _CH), mask.reshape(-1), emb_table,
            pos_table[:S], mask_table)
    return out.reshape(B, S, H)


# speedup vs baseline: 1.3329x; 1.0226x over previous
import functools

import jax
import jax.numpy as jnp
from jax import lax
from jax.experimental import pallas as pl
from jax.experimental.pallas import tpu as pltpu
from jax.experimental.pallas import tpu_sc as plsc

_CH = 128
_SCK = 256


def _make_kernel(B, S, H, V):
    info = plsc.get_sparse_core_info()
    NC, NS = info.num_cores, info.num_subcores
    NW = NC * NS
    TOK = B * S
    TPW = TOK // NW
    CH = _CH
    SCK = _SCK
    NSC = TPW // SCK
    NCH = TPW // CH
    G = H // 16
    NB = 3

    mesh = plsc.VectorSubcoreMesh(core_axis_name="c", subcore_axis_name="s")

    @functools.partial(
        pl.kernel,
        out_type=jax.ShapeDtypeStruct((TOK // CH, CH, H), jnp.float32),
        mesh=mesh,
        compiler_params=pltpu.CompilerParams(use_tc_tiling_on_sc=False),
        scratch_types=[
            pltpu.VMEM((NCH, CH), jnp.int32),
            pltpu.VMEM((TPW,), jnp.int32),
            pltpu.VMEM((2, CH, H), jnp.float32),
            pltpu.VMEM((2, CH, H), jnp.float32),
            pltpu.VMEM((2, CH, H), jnp.float32),
            pltpu.VMEM((S, H), jnp.float32),
            pltpu.VMEM((2 * S, H), jnp.float32),
            pltpu.VMEM((TPW + 16,), jnp.int32),
            pltpu.VMEM((2, H), jnp.float32),
            pltpu.SemaphoreType.DMA,
            pltpu.SemaphoreType.DMA,
            pltpu.SemaphoreType.DMA,
            pltpu.SemaphoreType.DMA,
            pltpu.SemaphoreType.DMA,
            pltpu.SemaphoreType.DMA,
        ],
    )
    def k(in_hbm, maskf_hbm, emb_hbm, pos_hbm, mt_hbm, out_hbm,
          tall, mall, erow0, erow1, erow2, posv, posm, cidx, mtv,
          semg0, semg1, semg2, semo0, semo1, semo2):
        wid = lax.axis_index("s") * NC + lax.axis_index("c")
        erow = (erow0, erow1, erow2)
        semg = (semg0, semg1, semg2)
        semo = (semo0, semo1, semo2)
        pltpu.sync_copy(in_hbm.at[pl.ds(wid * NCH, NCH), :], tall)
        for r in range(NB - 1):
            pltpu.async_copy(emb_hbm.at[tall.at[2 * r]],
                             erow[r].at[0], semg[r])
            pltpu.async_copy(emb_hbm.at[tall.at[2 * r + 1]],
                             erow[r].at[1], semg[r])
        pltpu.sync_copy(pos_hbm, posv)
        pltpu.sync_copy(mt_hbm, mtv)
        pltpu.sync_copy(maskf_hbm.at[pl.ds(wid * TPW, TPW)], mall)

        mt0 = [mtv[0, pl.ds(j * 16, 16)] for j in range(G)]
        mt1 = [mtv[1, pl.ds(j * 16, 16)] for j in range(G)]

        def pos_prep(s, carry):
            for j in range(G):
                sl = pl.ds(j * 16, 16)
                v = posv[s, sl]
                posm[2 * s, sl] = v + mt0[j]
                posm[2 * s + 1, sl] = v + mt1[j]
            return carry

        lax.fori_loop(0, S, pos_prep, 0)

        lane = jnp.arange(16, dtype=jnp.int32)

        def cidx_prep(ii, carry):
            sl = pl.ds(ii * 16, 16)
            cidx[sl] = 2 * lax.rem(lane + ii * 16, S) + mall[sl]
            return carry

        lax.fori_loop(0, TPW // 16, cidx_prep, 0)

        def fire_gather(u, p):
            pltpu.async_copy(emb_hbm.at[tall.at[2 * u]],
                             erow[p].at[0], semg[p])
            pltpu.async_copy(emb_hbm.at[tall.at[2 * u + 1]],
                             erow[p].at[1], semg[p])

        def out_slice(u):
            return out_hbm.at[pl.ds(wid * NCH + 2 * u, 2), :, :]

        def drain_gather(p):
            pltpu.make_async_copy(out_slice(0), erow[p], semg[p]).wait()

        def drain_out(u, p):
            pltpu.make_async_copy(erow[p], out_slice(u), semo[p]).wait()

        def compute(u, p):
            @plsc.parallel_loop(0, SCK, 1, unroll=8)
            def body(t):
                ci = cidx[pl.ds(u * SCK + t, 16)][0]
                h = t // CH
                r = t % CH
                for j in range(G):
                    sl = pl.ds(j * 16, 16)
                    erow[p][h, r, sl] = erow[p][h, r, sl] + posm[ci, sl]

        def stage(u, p):
            q = (p + NB - 1) % NB

            @pl.when(u < NSC)
            def _():
                @pl.when(u + NB - 1 < NSC)
                def _():
                    @pl.when(u >= 1)
                    def _():
                        drain_out(u - 1, q)
                    fire_gather(u + NB - 1, q)

                drain_gather(p)
                compute(u, p)
                pltpu.async_copy(erow[p], out_slice(u), semo[p])

        def ring_body(ii, carry):
            for r in range(NB):
                stage(NB * ii + r, r)
            return carry

        lax.fori_loop(0, (NSC + NB - 1) // NB, ring_body, 0)
        for u in range(NSC - NB, NSC):
            drain_out(u, u % NB)

    return k


def kernel(Input, mask, emb_table, pos_table, mask_table):
    B, S = Input.shape
    V, H = emb_table.shape
    k = _make_kernel(B, S, H, V)
    out = k(Input.reshape(-1, _CH), mask.reshape(-1), emb_table,
            pos_table[:S], mask_table)
    return out.reshape(B, S, H)
